# baseline (speedup 1.0000x reference)
import jax
import jax.numpy as jnp
from jax.experimental import pallas as pl
from jax.experimental.pallas import tpu as pltpu

_NB = 8192


def _match_kernel(q_ref, db_hbm, out_ref, db_v, sems):
    i = pl.program_id(0)
    nsteps = pl.num_programs(0)
    nb = db_v.shape[1]

    def _start(chunk, slot):
        pltpu.make_async_copy(
            db_hbm.at[pl.ds(chunk * nb, nb), :], db_v.at[slot], sems.at[slot]
        ).start()

    @pl.when(i == 0)
    def _():
        _start(0, 0)

    @pl.when(i + 1 < nsteps)
    def _():
        _start(i + 1, (i + 1) % 2)

    slot = i % 2
    pltpu.make_async_copy(
        db_hbm.at[pl.ds(i * nb, nb), :], db_v.at[slot], sems.at[slot]
    ).wait()

    sq = jnp.where(q_ref[...] > 0, 1.0, -1.0).astype(jnp.bfloat16)
    sdb = jnp.where(db_v[slot] > 0, 1.0, -1.0).astype(jnp.bfloat16)
    acc = jax.lax.dot_general(
        sq, sdb, (((1,), (1,)), ((), ())), preferred_element_type=jnp.float32
    )
    d = q_ref.shape[-1]
    out_ref[...] = (acc >= (d - 1.0)).astype(jnp.float32)


def kernel(queries, db):
    q, d = queries.shape
    n = db.shape[0]
    nb = _NB
    while n % nb:
        nb //= 2
    return pl.pallas_call(
        _match_kernel,
        grid=(n // nb,),
        in_specs=[
            pl.BlockSpec((q, d), lambda i: (0, 0)),
            pl.BlockSpec(memory_space=pl.ANY),
        ],
        out_specs=pl.BlockSpec((q, nb), lambda i: (0, i)),
        out_shape=jax.ShapeDtypeStruct((q, n), jnp.float32),
        scratch_shapes=[
            pltpu.VMEM((2, nb, d), jnp.float32),
            pltpu.SemaphoreType.DMA((2,)),
        ],
    )(queries, db)


# speedup vs baseline: 1.0408x; 1.0408x over previous
import jax
import jax.numpy as jnp
from jax.experimental import pallas as pl


def _match_kernel(q_ref, db_ref, out_ref):
    sq = jnp.where(q_ref[...] > 0, 1.0, -1.0).astype(jnp.bfloat16)
    sdb = jnp.where(db_ref[...] > 0, 1.0, -1.0).astype(jnp.bfloat16)
    acc = jax.lax.dot_general(
        sq, sdb, (((1,), (1,)), ((), ())), preferred_element_type=jnp.float32
    )
    d = q_ref.shape[-1]
    out_ref[...] = (acc >= (d - 1.0)).astype(jnp.float32)


def kernel(queries, db):
    q, d = queries.shape
    n = db.shape[0]
    nb = 16384
    while n % nb:
        nb //= 2
    return pl.pallas_call(
        _match_kernel,
        grid=(n // nb,),
        in_specs=[
            pl.BlockSpec((q, d), lambda i: (0, 0)),
            pl.BlockSpec((nb, d), lambda i: (i, 0)),
        ],
        out_specs=pl.BlockSpec((q, nb), lambda i: (0, i)),
        out_shape=jax.ShapeDtypeStruct((q, n), jnp.float32),
    )(queries, db)
